# baseline (speedup 1.0000x reference)
import functools

import jax
import jax.numpy as jnp
from jax import lax
from jax.experimental import pallas as pl
from jax.experimental.pallas import tpu as pltpu
from jax.experimental.pallas import tpu_sc as plsc

EPS = 1e-5
H = 128
C = 40
NC = 2
NS = 16
NW = NC * NS
LANES = 16


def _mm_body(x_ref, w_ref, b_ref, o_ref):
    o_ref[...] = (
        jnp.dot(x_ref[...], w_ref[...], preferred_element_type=jnp.float32)
        + b_ref[...]
    )


def _mm2_body(x_ref, w1_ref, b1_ref, w2_ref, b2_ref, o1_ref, o2_ref):
    y = (jnp.dot(x_ref[...], w1_ref[...], preferred_element_type=jnp.float32)
         + b1_ref[...])
    o1_ref[...] = y
    o2_ref[...] = (
        jnp.dot(y, w2_ref[...], preferred_element_type=jnp.float32)
        + b2_ref[...])


def _linear2(x, wt1, b1, wt2, b2, block_rows):
    rows, k = x.shape
    h1 = wt1.shape[1]
    h2 = wt2.shape[1]
    assert rows % block_rows == 0
    return pl.pallas_call(
        _mm2_body,
        grid=(rows // block_rows,),
        in_specs=[
            pl.BlockSpec((block_rows, k), lambda i: (i, 0)),
            pl.BlockSpec((k, h1), lambda i: (0, 0)),
            pl.BlockSpec((1, h1), lambda i: (0, 0)),
            pl.BlockSpec((h1, h2), lambda i: (0, 0)),
            pl.BlockSpec((1, h2), lambda i: (0, 0)),
        ],
        out_specs=[
            pl.BlockSpec((block_rows, h1), lambda i: (i, 0)),
            pl.BlockSpec((block_rows, h2), lambda i: (i, 0)),
        ],
        out_shape=[
            jax.ShapeDtypeStruct((rows, h1), jnp.float32),
            jax.ShapeDtypeStruct((rows, h2), jnp.float32),
        ],
    )(x, wt1, b1.reshape(1, -1), wt2, b2.reshape(1, -1))


def _linear(x, wt, b, block_rows):
    rows, k = x.shape
    h = wt.shape[1]
    assert rows % block_rows == 0
    return pl.pallas_call(
        _mm_body,
        grid=(rows // block_rows,),
        in_specs=[
            pl.BlockSpec((block_rows, k), lambda i: (i, 0)),
            pl.BlockSpec((k, h), lambda i: (0, 0)),
            pl.BlockSpec((1, h), lambda i: (0, 0)),
        ],
        out_specs=pl.BlockSpec((block_rows, h), lambda i: (i, 0)),
        out_shape=jax.ShapeDtypeStruct((rows, h), jnp.float32),
    )(x, wt, b.reshape(1, -1))


def _sc_edge_body(need_e, nch, ce_hbm, dh_hbm, eh_hbm, bh_hbm, src_hbm,
                  dst_hbm, zeros_hbm, ehat_hbm, stats_hbm, aggp_hbm,
                  idx_s0, idx_d0, ce_v0, dh_v0, eh_v0, bh_v0,
                  idx_s1, idx_d1, ce_v1, dh_v1, eh_v1, bh_v1,
                  sum_v, sq_v, agg_sh,
                  sem_idx0, sem_idx1, sem_in0, sem_in1, sem_out0, sem_out1):
    c = lax.axis_index("c")
    s = lax.axis_index("s")
    wid = s * NC + c

    bufs = [(idx_s0, idx_d0, ce_v0, dh_v0, eh_v0, bh_v0),
            (idx_s1, idx_d1, ce_v1, dh_v1, eh_v1, bh_v1)]
    sem_idx = [sem_idx0, sem_idx1]
    sem_in = [sem_in0, sem_in1]
    sem_out = [sem_out0, sem_out1]

    @pl.when(s == 0)
    def _zero():
        pltpu.sync_copy(zeros_hbm, agg_sh)

    plsc.subcore_barrier()

    assert nch % NW == 0 and (nch // NW) % 2 == 0
    npairs = (nch // NW) // 2

    def _base(k):
        return (wid + k * NW) * C

    def issue_idx(k, b):
        idx_s, idx_d = bufs[b][0], bufs[b][1]
        base = _base(k)
        pltpu.async_copy(src_hbm.at[pl.ds(base, C)], idx_s, sem_idx[b])
        pltpu.async_copy(dst_hbm.at[pl.ds(base, C)], idx_d, sem_idx[b])

    def wait_idx(b):
        idx_s, idx_d = bufs[b][0], bufs[b][1]
        pltpu.make_async_copy(src_hbm.at[pl.ds(0, C)], idx_s, sem_idx[b]).wait()
        pltpu.make_async_copy(dst_hbm.at[pl.ds(0, C)], idx_d, sem_idx[b]).wait()

    def issue_in(k, b):
        idx_s, idx_d, ce_v, dh_v, eh_v, bh_v = bufs[b]
        base = _base(k)
        pltpu.async_copy(ce_hbm.at[pl.ds(base, C)], ce_v, sem_in[b])
        pltpu.async_copy(dh_hbm.at[idx_d], dh_v, sem_in[b])
        pltpu.async_copy(eh_hbm.at[idx_s], eh_v, sem_in[b])
        pltpu.async_copy(bh_hbm.at[idx_s], bh_v, sem_in[b])

    def wait_in(b):
        idx_s, idx_d, ce_v, dh_v, eh_v, bh_v = bufs[b]
        pltpu.make_async_copy(ce_hbm.at[pl.ds(0, C)], ce_v, sem_in[b]).wait()
        pltpu.make_async_copy(dh_hbm.at[idx_d], dh_v, sem_in[b]).wait()
        pltpu.make_async_copy(eh_hbm.at[idx_s], eh_v, sem_in[b]).wait()
        pltpu.make_async_copy(bh_hbm.at[idx_s], bh_v, sem_in[b]).wait()

    def issue_out(k, b):
        idx_d, ce_v, bh_v = bufs[b][1], bufs[b][2], bufs[b][5]
        if need_e:
            pltpu.async_copy(ce_v, ehat_hbm.at[pl.ds(_base(k), C)], sem_out[b])
        pltpu.sync_copy(bh_v, agg_sh.at[idx_d], add=True)

    def wait_out(b):
        ce_v = bufs[b][2]
        if need_e:
            pltpu.make_async_copy(
                ce_v, ehat_hbm.at[pl.ds(0, C)], sem_out[b]).wait()

    def compute(b, acc):
        ce_v, dh_v, eh_v, bh_v = bufs[b][2], bufs[b][3], bufs[b][4], bufs[b][5]

        def row_body(r, racc):
            new_s = []
            new_q = []
            for j in range(H // LANES):
                sl = pl.ds(j * LANES, LANES)
                ehat = ce_v[r, sl] + dh_v[r, sl] + eh_v[r, sl]
                sig = 1.0 / (1.0 + jnp.exp(-ehat))
                bh_v[r, sl] = sig * bh_v[r, sl]
                if need_e:
                    ce_v[r, sl] = ehat
                    new_s.append(racc[j] + ehat)
                    new_q.append(racc[j + H // LANES] + ehat * ehat)
            return tuple(new_s + new_q) if need_e else racc

        return lax.fori_loop(0, C, row_body, acc)

    issue_idx(0, 0)
    issue_idx(1, 1)
    wait_idx(0)
    issue_in(0, 0)

    def pair_body(i, acc):
        @pl.when(i > 0)
        def _():
            wait_out(1)
        wait_idx(1)
        issue_in(2 * i + 1, 1)
        wait_in(0)
        acc = compute(0, acc)
        issue_out(2 * i, 0)

        @pl.when(i < npairs - 1)
        def _():
            issue_idx(2 * i + 2, 0)

        @pl.when(i < npairs - 1)
        def _():
            wait_out(0)
            wait_idx(0)
            issue_in(2 * i + 2, 0)

        wait_in(1)
        acc = compute(1, acc)
        issue_out(2 * i + 1, 1)

        @pl.when(i < npairs - 1)
        def _():
            issue_idx(2 * i + 3, 1)

        return acc

    zero_acc = tuple(
        jnp.zeros((LANES,), jnp.float32) for _ in range(2 * (H // LANES))
    )
    acc = lax.fori_loop(0, npairs, pair_body, zero_acc)
    wait_out(0)
    wait_out(1)

    if need_e:
        for j in range(H // LANES):
            sum_v[pl.ds(j * LANES, LANES)] = acc[j]
            sq_v[pl.ds(j * LANES, LANES)] = acc[j + H // LANES]
        pltpu.sync_copy(sum_v, stats_hbm.at[0, pl.ds(wid * H, H)])
        pltpu.sync_copy(sq_v, stats_hbm.at[1, pl.ds(wid * H, H)])

    plsc.subcore_barrier()
    n = zeros_hbm.shape[0]
    stripe = ((n + NS - 1) // NS + 7) // 8 * 8
    tail = n - (NS - 1) * stripe

    @pl.when(s < NS - 1)
    def _copy_full():
        pltpu.sync_copy(
            agg_sh.at[pl.ds(s * stripe, stripe)],
            aggp_hbm.at[c, pl.ds(s * stripe, stripe)],
        )

    @pl.when(s == NS - 1)
    def _copy_tail():
        pltpu.sync_copy(
            agg_sh.at[pl.ds((NS - 1) * stripe, tail)],
            aggp_hbm.at[c, pl.ds((NS - 1) * stripe, tail)],
        )


@functools.partial(jax.jit, static_argnames=("need_e",))
def _sc_edge(ce, dh, eh, bh, src, dst, zeros, need_e):
    e_edges = ce.shape[0]
    n = dh.shape[0]
    assert e_edges % C == 0 and n % NS == 0
    nch = e_edges // C
    mesh = plsc.VectorSubcoreMesh(
        core_axis_name="c", subcore_axis_name="s", num_cores=NC,
        num_subcores=NS,
    )
    out_type = [
        jax.ShapeDtypeStruct((e_edges, H) if need_e else (1, H), jnp.float32),
        jax.ShapeDtypeStruct((2, NW * H) if need_e else (1, H), jnp.float32),
        jax.ShapeDtypeStruct((NC, n, H), jnp.float32),
    ]
    scratch = (
        [pltpu.VMEM((C,), jnp.int32),
         pltpu.VMEM((C,), jnp.int32),
         pltpu.VMEM((C, H), jnp.float32),
         pltpu.VMEM((C, H), jnp.float32),
         pltpu.VMEM((C, H), jnp.float32),
         pltpu.VMEM((C, H), jnp.float32)] * 2
        + [pltpu.VMEM((H,), jnp.float32),
           pltpu.VMEM((H,), jnp.float32),
           pltpu.VMEM_SHARED((n, H), jnp.float32)]
        + [pltpu.SemaphoreType.DMA] * 6
    )
    fn = pl.kernel(
        functools.partial(_sc_edge_body, need_e, nch),
        out_type=out_type,
        mesh=mesh,
        scratch_types=scratch,
    )
    return fn(ce, dh, eh, bh, src, dst, zeros)


def _edge_apply_mm_body(ne, stats_ref, g_ref, b_ref, e_ref, ehat_ref,
                        w_ref, cb_ref, o_ref):
    ssum = jnp.sum(stats_ref[0], axis=0)
    ssq = jnp.sum(stats_ref[1], axis=0)
    m = ssum / ne
    v = ssq / ne - m * m
    scale = g_ref[0] / jnp.sqrt(v + EPS)
    shift = b_ref[0] - m * scale
    x = e_ref[...] + jnp.maximum(ehat_ref[...] * scale + shift, 0.0)
    o_ref[...] = (
        jnp.dot(x, w_ref[...], preferred_element_type=jnp.float32)
        + cb_ref[...])


def _edge_apply_mm(stats, g, b, e, ehat, wt, cb, block_rows):
    rows = e.shape[0]
    h2 = wt.shape[1]
    assert rows % block_rows == 0
    return pl.pallas_call(
        functools.partial(_edge_apply_mm_body, float(rows)),
        grid=(rows // block_rows,),
        in_specs=[
            pl.BlockSpec(stats.shape, lambda i: (0, 0, 0)),
            pl.BlockSpec((1, H), lambda i: (0, 0)),
            pl.BlockSpec((1, H), lambda i: (0, 0)),
            pl.BlockSpec((block_rows, H), lambda i: (i, 0)),
            pl.BlockSpec((block_rows, H), lambda i: (i, 0)),
            pl.BlockSpec((H, h2), lambda i: (0, 0)),
            pl.BlockSpec((1, h2), lambda i: (0, 0)),
        ],
        out_specs=pl.BlockSpec((block_rows, h2), lambda i: (i, 0)),
        out_shape=jax.ShapeDtypeStruct((rows, h2), jnp.float32),
    )(stats, g.reshape(1, -1), b.reshape(1, -1), e, ehat, wt,
      cb.reshape(1, -1))


def _node_apply_body(ah_ref, aggp_ref, h_ref, g_ref, b_ref, o_ref):
    x = ah_ref[...] + aggp_ref[0] + aggp_ref[1]
    m = jnp.mean(x, axis=0)
    v = jnp.mean(x * x, axis=0) - m * m
    o_ref[...] = h_ref[...] + jnp.maximum(
        (x - m) / jnp.sqrt(v + EPS) * g_ref[0] + b_ref[0], 0.0)


def _node_apply(ah, aggp, h, g, b):
    n = ah.shape[0]
    return pl.pallas_call(
        _node_apply_body,
        out_shape=jax.ShapeDtypeStruct((n, H), jnp.float32),
    )(ah, aggp, h, g.reshape(1, -1), b.reshape(1, -1))


def _final_body(h_ref, w1_ref, b1_ref, w2_ref, b2_ref, o_ref):
    m = jnp.mean(h_ref[...], axis=0, keepdims=True)
    y = jnp.maximum(
        jnp.dot(m, w1_ref[...], preferred_element_type=jnp.float32)
        + b1_ref[...], 0.0)
    o_ref[...] = (
        jnp.dot(y, w2_ref[...], preferred_element_type=jnp.float32)
        + b2_ref[...])


def _final(h, w1t, b1, w2t, b2):
    out = w2t.shape[1]
    return pl.pallas_call(
        _final_body,
        out_shape=jax.ShapeDtypeStruct((1, out), jnp.float32),
    )(h, w1t, b1.reshape(1, -1), w2t, b2.reshape(1, -1))


def kernel(h, edge_index, edge_attr, params):
    p = params
    n = h.shape[0]
    e_edges = edge_attr.shape[0]
    nblk = max(1, n // 5)
    eblk = max(1, e_edges // 80)
    num_layers = p['A_W'].shape[0]
    src = edge_index[0]
    dst = edge_index[1]
    zeros = jnp.zeros((n, H), jnp.float32)

    hh = _linear(h, p['node_W'].T, p['node_b'], nblk)
    e, ce = _linear2(edge_attr, p['edge_W'].T, p['edge_b'],
                     p['C_W'][0].T, p['C_b'][0], eblk)

    prev = None
    for l in range(num_layers):
        ah = _linear(hh, p['A_W'][l].T, p['A_b'][l], nblk)
        dh = _linear(hh, p['D_W'][l].T, p['D_b'][l], nblk)
        eh = _linear(hh, p['E_W'][l].T, p['E_b'][l], nblk)
        bh = _linear(hh, p['B_W'][l].T, p['B_b'][l], nblk)
        if l > 0:
            stats, ehat = prev
            ce = _edge_apply_mm(stats.reshape(2, NW, H),
                                p['bn_edge_g'][l - 1], p['bn_edge_b'][l - 1],
                                e, ehat, p['C_W'][l].T, p['C_b'][l], eblk)
        need_e = l < num_layers - 1
        ehat, stats, aggp = _sc_edge(ce, dh, eh, bh, src, dst, zeros,
                                     need_e=need_e)
        prev = (stats, ehat)
        hh = _node_apply(ah, aggp, hh, p['bn_node_g'][l], p['bn_node_b'][l])

    return _final(hh, p['cls_W1'].T, p['cls_b1'], p['cls_W2'].T, p['cls_b2'])


# speedup vs baseline: 1.0245x; 1.0245x over previous
import functools

import jax
import jax.numpy as jnp
from jax import lax
from jax.experimental import pallas as pl
from jax.experimental.pallas import tpu as pltpu
from jax.experimental.pallas import tpu_sc as plsc

EPS = 1e-5
H = 128
C = 40
NC = 2
NS = 16
NW = NC * NS
LANES = 16


def _mm_body(x_ref, w_ref, b_ref, o_ref):
    o_ref[...] = (
        jnp.dot(x_ref[...], w_ref[...], preferred_element_type=jnp.float32)
        + b_ref[...]
    )


def _mm2_body(x_ref, w1_ref, b1_ref, w2_ref, b2_ref, o1_ref, o2_ref):
    y = (jnp.dot(x_ref[...], w1_ref[...], preferred_element_type=jnp.float32)
         + b1_ref[...])
    o1_ref[...] = y.astype(o1_ref.dtype)
    o2_ref[...] = (
        jnp.dot(y, w2_ref[...], preferred_element_type=jnp.float32)
        + b2_ref[...])


def _linear2(x, wt1, b1, wt2, b2, block_rows):
    rows, k = x.shape
    h1 = wt1.shape[1]
    h2 = wt2.shape[1]
    assert rows % block_rows == 0
    return pl.pallas_call(
        _mm2_body,
        grid=(rows // block_rows,),
        in_specs=[
            pl.BlockSpec((block_rows, k), lambda i: (i, 0)),
            pl.BlockSpec((k, h1), lambda i: (0, 0)),
            pl.BlockSpec((1, h1), lambda i: (0, 0)),
            pl.BlockSpec((h1, h2), lambda i: (0, 0)),
            pl.BlockSpec((1, h2), lambda i: (0, 0)),
        ],
        out_specs=[
            pl.BlockSpec((block_rows, h1), lambda i: (i, 0)),
            pl.BlockSpec((block_rows, h2), lambda i: (i, 0)),
        ],
        out_shape=[
            jax.ShapeDtypeStruct((rows, h1), jnp.bfloat16),
            jax.ShapeDtypeStruct((rows, h2), jnp.float32),
        ],
    )(x, wt1, b1.reshape(1, -1), wt2, b2.reshape(1, -1))


def _linear(x, wt, b, block_rows):
    rows, k = x.shape
    h = wt.shape[1]
    assert rows % block_rows == 0
    return pl.pallas_call(
        _mm_body,
        grid=(rows // block_rows,),
        in_specs=[
            pl.BlockSpec((block_rows, k), lambda i: (i, 0)),
            pl.BlockSpec((k, h), lambda i: (0, 0)),
            pl.BlockSpec((1, h), lambda i: (0, 0)),
        ],
        out_specs=pl.BlockSpec((block_rows, h), lambda i: (i, 0)),
        out_shape=jax.ShapeDtypeStruct((rows, h), jnp.float32),
    )(x, wt, b.reshape(1, -1))


def _sc_edge_body(need_e, nch, ce_hbm, dh_hbm, eh_hbm, bh_hbm, src_hbm,
                  dst_hbm, zeros_hbm, ehat_hbm, stats_hbm, aggp_hbm,
                  idx_s0, idx_d0, ce_v0, dh_v0, eh_v0, bh_v0,
                  idx_s1, idx_d1, ce_v1, dh_v1, eh_v1, bh_v1,
                  sum_v, sq_v, agg_sh,
                  sem_idx0, sem_idx1, sem_in0, sem_in1, sem_out0, sem_out1):
    c = lax.axis_index("c")
    s = lax.axis_index("s")
    wid = s * NC + c

    bufs = [(idx_s0, idx_d0, ce_v0, dh_v0, eh_v0, bh_v0),
            (idx_s1, idx_d1, ce_v1, dh_v1, eh_v1, bh_v1)]
    sem_idx = [sem_idx0, sem_idx1]
    sem_in = [sem_in0, sem_in1]
    sem_out = [sem_out0, sem_out1]

    @pl.when(s == 0)
    def _zero():
        pltpu.sync_copy(zeros_hbm, agg_sh)

    plsc.subcore_barrier()

    assert nch % NW == 0 and (nch // NW) % 2 == 0
    npairs = (nch // NW) // 2

    def _base(k):
        return (wid + k * NW) * C

    def issue_idx(k, b):
        idx_s, idx_d = bufs[b][0], bufs[b][1]
        base = _base(k)
        pltpu.async_copy(src_hbm.at[pl.ds(base, C)], idx_s, sem_idx[b])
        pltpu.async_copy(dst_hbm.at[pl.ds(base, C)], idx_d, sem_idx[b])

    def wait_idx(b):
        idx_s, idx_d = bufs[b][0], bufs[b][1]
        pltpu.make_async_copy(src_hbm.at[pl.ds(0, C)], idx_s, sem_idx[b]).wait()
        pltpu.make_async_copy(dst_hbm.at[pl.ds(0, C)], idx_d, sem_idx[b]).wait()

    def issue_in(k, b):
        idx_s, idx_d, ce_v, dh_v, eh_v, bh_v = bufs[b]
        base = _base(k)
        pltpu.async_copy(ce_hbm.at[pl.ds(base, C)], ce_v, sem_in[b])
        pltpu.async_copy(dh_hbm.at[idx_d], dh_v, sem_in[b])
        pltpu.async_copy(eh_hbm.at[idx_s], eh_v, sem_in[b])
        pltpu.async_copy(bh_hbm.at[idx_s], bh_v, sem_in[b])

    def wait_in(b):
        idx_s, idx_d, ce_v, dh_v, eh_v, bh_v = bufs[b]
        pltpu.make_async_copy(ce_hbm.at[pl.ds(0, C)], ce_v, sem_in[b]).wait()
        pltpu.make_async_copy(dh_hbm.at[idx_d], dh_v, sem_in[b]).wait()
        pltpu.make_async_copy(eh_hbm.at[idx_s], eh_v, sem_in[b]).wait()
        pltpu.make_async_copy(bh_hbm.at[idx_s], bh_v, sem_in[b]).wait()

    def issue_out(k, b):
        idx_d, ce_v, bh_v = bufs[b][1], bufs[b][2], bufs[b][5]
        if need_e:
            pltpu.async_copy(ce_v, ehat_hbm.at[pl.ds(_base(k), C)], sem_out[b])
        pltpu.sync_copy(bh_v, agg_sh.at[idx_d], add=True)

    def wait_out(b):
        ce_v = bufs[b][2]
        if need_e:
            pltpu.make_async_copy(
                ce_v, ehat_hbm.at[pl.ds(0, C)], sem_out[b]).wait()

    def compute(b, acc):
        ce_v, dh_v, eh_v, bh_v = bufs[b][2], bufs[b][3], bufs[b][4], bufs[b][5]

        def row_body(r, racc):
            new_s = []
            new_q = []
            for j in range(H // LANES):
                sl = pl.ds(j * LANES, LANES)
                ehat = ce_v[r, sl] + dh_v[r, sl] + eh_v[r, sl]
                sig = 1.0 / (1.0 + jnp.exp(-ehat))
                bh_v[r, sl] = sig * bh_v[r, sl]
                if need_e:
                    ce_v[r, sl] = ehat
                    new_s.append(racc[j] + ehat)
                    new_q.append(racc[j + H // LANES] + ehat * ehat)
            return tuple(new_s + new_q) if need_e else racc

        return lax.fori_loop(0, C, row_body, acc)

    issue_idx(0, 0)
    issue_idx(1, 1)
    wait_idx(0)
    issue_in(0, 0)

    def pair_body(i, acc):
        @pl.when(i > 0)
        def _():
            wait_out(1)
        wait_idx(1)
        issue_in(2 * i + 1, 1)
        wait_in(0)
        acc = compute(0, acc)
        issue_out(2 * i, 0)

        @pl.when(i < npairs - 1)
        def _():
            issue_idx(2 * i + 2, 0)

        @pl.when(i < npairs - 1)
        def _():
            wait_out(0)
            wait_idx(0)
            issue_in(2 * i + 2, 0)

        wait_in(1)
        acc = compute(1, acc)
        issue_out(2 * i + 1, 1)

        @pl.when(i < npairs - 1)
        def _():
            issue_idx(2 * i + 3, 1)

        return acc

    zero_acc = tuple(
        jnp.zeros((LANES,), jnp.float32) for _ in range(2 * (H // LANES))
    )
    acc = lax.fori_loop(0, npairs, pair_body, zero_acc)
    wait_out(0)
    wait_out(1)

    if need_e:
        for j in range(H // LANES):
            sum_v[pl.ds(j * LANES, LANES)] = acc[j]
            sq_v[pl.ds(j * LANES, LANES)] = acc[j + H // LANES]
        pltpu.sync_copy(sum_v, stats_hbm.at[0, pl.ds(wid * H, H)])
        pltpu.sync_copy(sq_v, stats_hbm.at[1, pl.ds(wid * H, H)])

    plsc.subcore_barrier()
    n = zeros_hbm.shape[0]
    stripe = ((n + NS - 1) // NS + 7) // 8 * 8
    tail = n - (NS - 1) * stripe

    @pl.when(s < NS - 1)
    def _copy_full():
        pltpu.sync_copy(
            agg_sh.at[pl.ds(s * stripe, stripe)],
            aggp_hbm.at[c, pl.ds(s * stripe, stripe)],
        )

    @pl.when(s == NS - 1)
    def _copy_tail():
        pltpu.sync_copy(
            agg_sh.at[pl.ds((NS - 1) * stripe, tail)],
            aggp_hbm.at[c, pl.ds((NS - 1) * stripe, tail)],
        )


@functools.partial(jax.jit, static_argnames=("need_e",))
def _sc_edge(ce, dh, eh, bh, src, dst, zeros, need_e):
    e_edges = ce.shape[0]
    n = dh.shape[0]
    assert e_edges % C == 0 and n % NS == 0
    nch = e_edges // C
    mesh = plsc.VectorSubcoreMesh(
        core_axis_name="c", subcore_axis_name="s", num_cores=NC,
        num_subcores=NS,
    )
    out_type = [
        jax.ShapeDtypeStruct((e_edges, H) if need_e else (1, H), jnp.float32),
        jax.ShapeDtypeStruct((2, NW * H) if need_e else (1, H), jnp.float32),
        jax.ShapeDtypeStruct((NC, n, H), jnp.float32),
    ]
    scratch = (
        [pltpu.VMEM((C,), jnp.int32),
         pltpu.VMEM((C,), jnp.int32),
         pltpu.VMEM((C, H), jnp.float32),
         pltpu.VMEM((C, H), jnp.float32),
         pltpu.VMEM((C, H), jnp.float32),
         pltpu.VMEM((C, H), jnp.float32)] * 2
        + [pltpu.VMEM((H,), jnp.float32),
           pltpu.VMEM((H,), jnp.float32),
           pltpu.VMEM_SHARED((n, H), jnp.float32)]
        + [pltpu.SemaphoreType.DMA] * 6
    )
    fn = pl.kernel(
        functools.partial(_sc_edge_body, need_e, nch),
        out_type=out_type,
        mesh=mesh,
        scratch_types=scratch,
    )
    return fn(ce, dh, eh, bh, src, dst, zeros)


def _edge_apply_mm_body(ne, stats_ref, g_ref, b_ref, e_ref, ehat_ref,
                        w_ref, cb_ref, o_ref):
    ssum = jnp.sum(stats_ref[0], axis=0)
    ssq = jnp.sum(stats_ref[1], axis=0)
    m = ssum / ne
    v = ssq / ne - m * m
    scale = g_ref[0] / jnp.sqrt(v + EPS)
    shift = b_ref[0] - m * scale
    x = e_ref[...].astype(jnp.float32) + jnp.maximum(
        ehat_ref[...] * scale + shift, 0.0)
    o_ref[...] = (
        jnp.dot(x, w_ref[...], preferred_element_type=jnp.float32)
        + cb_ref[...])


def _edge_apply_mm(stats, g, b, e, ehat, wt, cb, block_rows):
    rows = e.shape[0]
    h2 = wt.shape[1]
    assert rows % block_rows == 0
    return pl.pallas_call(
        functools.partial(_edge_apply_mm_body, float(rows)),
        grid=(rows // block_rows,),
        in_specs=[
            pl.BlockSpec(stats.shape, lambda i: (0, 0, 0)),
            pl.BlockSpec((1, H), lambda i: (0, 0)),
            pl.BlockSpec((1, H), lambda i: (0, 0)),
            pl.BlockSpec((block_rows, H), lambda i: (i, 0)),
            pl.BlockSpec((block_rows, H), lambda i: (i, 0)),
            pl.BlockSpec((H, h2), lambda i: (0, 0)),
            pl.BlockSpec((1, h2), lambda i: (0, 0)),
        ],
        out_specs=pl.BlockSpec((block_rows, h2), lambda i: (i, 0)),
        out_shape=jax.ShapeDtypeStruct((rows, h2), jnp.float32),
    )(stats, g.reshape(1, -1), b.reshape(1, -1), e, ehat, wt,
      cb.reshape(1, -1))


def _node_apply_body(ah_ref, aggp_ref, h_ref, g_ref, b_ref, o_ref):
    x = ah_ref[...] + aggp_ref[0] + aggp_ref[1]
    m = jnp.mean(x, axis=0)
    v = jnp.mean(x * x, axis=0) - m * m
    o_ref[...] = h_ref[...] + jnp.maximum(
        (x - m) / jnp.sqrt(v + EPS) * g_ref[0] + b_ref[0], 0.0)


def _node_apply(ah, aggp, h, g, b):
    n = ah.shape[0]
    return pl.pallas_call(
        _node_apply_body,
        out_shape=jax.ShapeDtypeStruct((n, H), jnp.float32),
    )(ah, aggp, h, g.reshape(1, -1), b.reshape(1, -1))


def _final_body(h_ref, w1_ref, b1_ref, w2_ref, b2_ref, o_ref):
    m = jnp.mean(h_ref[...], axis=0, keepdims=True)
    y = jnp.maximum(
        jnp.dot(m, w1_ref[...], preferred_element_type=jnp.float32)
        + b1_ref[...], 0.0)
    o_ref[...] = (
        jnp.dot(y, w2_ref[...], preferred_element_type=jnp.float32)
        + b2_ref[...])


def _final(h, w1t, b1, w2t, b2):
    out = w2t.shape[1]
    return pl.pallas_call(
        _final_body,
        out_shape=jax.ShapeDtypeStruct((1, out), jnp.float32),
    )(h, w1t, b1.reshape(1, -1), w2t, b2.reshape(1, -1))


def kernel(h, edge_index, edge_attr, params):
    p = params
    n = h.shape[0]
    e_edges = edge_attr.shape[0]
    nblk = max(1, n // 5)
    eblk = max(1, e_edges // 80)
    num_layers = p['A_W'].shape[0]
    src = edge_index[0]
    dst = edge_index[1]
    zeros = jnp.zeros((n, H), jnp.float32)

    hh = _linear(h, p['node_W'].T, p['node_b'], nblk)
    e, ce = _linear2(edge_attr, p['edge_W'].T, p['edge_b'],
                     p['C_W'][0].T, p['C_b'][0], eblk)

    prev = None
    for l in range(num_layers):
        ah = _linear(hh, p['A_W'][l].T, p['A_b'][l], nblk)
        dh = _linear(hh, p['D_W'][l].T, p['D_b'][l], nblk)
        eh = _linear(hh, p['E_W'][l].T, p['E_b'][l], nblk)
        bh = _linear(hh, p['B_W'][l].T, p['B_b'][l], nblk)
        if l > 0:
            stats, ehat = prev
            ce = _edge_apply_mm(stats.reshape(2, NW, H),
                                p['bn_edge_g'][l - 1], p['bn_edge_b'][l - 1],
                                e, ehat, p['C_W'][l].T, p['C_b'][l], eblk)
        need_e = l < num_layers - 1
        ehat, stats, aggp = _sc_edge(ce, dh, eh, bh, src, dst, zeros,
                                     need_e=need_e)
        prev = (stats, ehat)
        hh = _node_apply(ah, aggp, hh, p['bn_node_g'][l], p['bn_node_b'][l])

    return _final(hh, p['cls_W1'].T, p['cls_b1'], p['cls_W2'].T, p['cls_b2'])
